# baseline (speedup 1.0000x reference)
import functools

import jax
import jax.numpy as jnp
from jax import lax
from jax.experimental import pallas as pl
from jax.experimental.pallas import tpu as pltpu
from jax.experimental.pallas import tpu_sc as plsc

B = 1024
L = 200
D = 32
NUM_RELATIONS = 1000

NC = 2
NS = 16
NW = NC * NS
N = B * L
ROWS_PER_W = N // NW
CHUNK = 128
NCHUNK = ROWS_PER_W // CHUNK
LANES = 16
GROUPS = CHUNK // LANES
PACK = 128 // D


def _sc_body(ent_hbm, srel_hbm, nidx_hbm, cmb_hbm, out_hbm,
             srel_v, nbs, cms, be0, be1, ob0, ob1, sem0, sem1, so0, so1):
    out_hbm = out_hbm.reshape(N, D)
    wid = lax.axis_index("s") * NC + lax.axis_index("c")
    obase = wid * ROWS_PER_W

    pltpu.sync_copy(srel_hbm, srel_v)
    pltpu.sync_copy(nidx_hbm.at[wid], nbs)
    pltpu.sync_copy(cmb_hbm.at[wid], cms)

    def to_lines(k, carry):
        for g in range(GROUPS):
            sl = pl.ds(g * LANES, LANES)
            nbs[k, sl] = lax.shift_right_logical(nbs[k, sl], 2)
        return carry

    lax.fori_loop(0, NCHUNK, to_lines, 0)

    def stage(k, be, sem):
        pltpu.async_copy(ent_hbm.at[nbs.at[k]], be, sem)

    def out_dma(k, ob, so):
        return pltpu.make_async_copy(
            ob, out_hbm.at[pl.ds(obase + k * CHUNK, CHUNK)], so)

    def process(k, be, ob, sem, so):
        pltpu.make_async_copy(ent_hbm.at[nbs.at[k]], be, sem).wait()
        for g in range(GROUPS):
            cvv = cms[k, pl.ds(g * LANES, LANES)]
            for i in range(LANES):
                r = g * LANES + i
                c = cvv[i]
                o = c & 127
                s = lax.shift_right_logical(c, 8)
                sgn = jnp.where((c & 128) == 0, jnp.float32(-1), jnp.float32(1))
                for cb in range(0, D, LANES):
                    ev = be[r, pl.ds(o + cb, LANES)]
                    rv = srel_v[pl.ds(s + cb, LANES)]
                    ob[r, pl.ds(cb, LANES)] = ev + rv * sgn
        pltpu.async_copy(ob, out_hbm.at[pl.ds(obase + k * CHUNK, CHUNK)], so)

    stage(0, be0, sem0)

    def pair(i, carry):
        k = i * 2
        stage(k + 1, be1, sem1)

        @pl.when(k > 0)
        def _():
            out_dma(k - 2, ob0, so0).wait()

        process(k, be0, ob0, sem0, so0)

        @pl.when(k + 2 < NCHUNK)
        def _():
            stage(k + 2, be0, sem0)

        @pl.when(k > 0)
        def _():
            out_dma(k - 1, ob1, so1).wait()

        process(k + 1, be1, ob1, sem1, so1)
        return carry

    lax.fori_loop(0, NCHUNK // 2, pair, 0)
    out_dma(NCHUNK - 2, ob0, so0).wait()
    out_dma(NCHUNK - 1, ob1, so1).wait()


@functools.partial(
    pl.kernel,
    out_type=jax.ShapeDtypeStruct((B, L, D), jnp.float32),
    mesh=plsc.VectorSubcoreMesh(
        core_axis_name="c", subcore_axis_name="s",
        num_cores=NC, num_subcores=NS),
    scratch_types=[
        pltpu.VMEM((NUM_RELATIONS * D,), jnp.float32),
        pltpu.VMEM((NCHUNK, CHUNK), jnp.int32),
        pltpu.VMEM((NCHUNK, CHUNK), jnp.int32),
        pltpu.VMEM((CHUNK, 128), jnp.float32),
        pltpu.VMEM((CHUNK, 128), jnp.float32),
        pltpu.VMEM((CHUNK, D), jnp.float32),
        pltpu.VMEM((CHUNK, D), jnp.float32),
        pltpu.SemaphoreType.DMA,
        pltpu.SemaphoreType.DMA,
        pltpu.SemaphoreType.DMA,
        pltpu.SemaphoreType.DMA,
    ],
)
def _transe_sc(ent_hbm, srel_hbm, nidx_hbm, cmb_hbm, out_hbm,
               srel_v, nbs, cms, be0, be1, ob0, ob1, sem0, sem1, so0, so1):
    _sc_body(ent_hbm, srel_hbm, nidx_hbm, cmb_hbm, out_hbm,
             srel_v, nbs, cms, be0, be1, ob0, ob1, sem0, sem1, so0, so1)


def kernel(entity_table, relation_table, neigh_nodes, neigh_rels, inverse):
    ent4 = entity_table.reshape(1000000 // PACK, 128)
    srel = relation_table.reshape(-1)
    n32 = neigh_nodes.astype(jnp.int32)
    nidx = n32.reshape(NW, NCHUNK, CHUNK)
    lane_off = (n32 & (PACK - 1)) << 5
    rel_off = neigh_rels.astype(jnp.int32) << 5
    cmb = ((rel_off << 8) | (inverse.astype(jnp.int32) << 7)
           | lane_off).reshape(NW, NCHUNK, CHUNK)
    return _transe_sc(ent4, srel, nidx, cmb)


# speedup vs baseline: 1.0805x; 1.0805x over previous
import functools

import jax
import jax.numpy as jnp
from jax import lax
from jax.experimental import pallas as pl
from jax.experimental.pallas import tpu as pltpu
from jax.experimental.pallas import tpu_sc as plsc

B = 1024
L = 200
D = 32
NUM_RELATIONS = 1000

NC = 2
NS = 16
NW = NC * NS
N = B * L
ROWS_PER_W = N // NW
CHUNK = 128
NCHUNK = ROWS_PER_W // CHUNK
LANES = 16
GROUPS = CHUNK // LANES
PACK = 128 // D


def _sc_body(ent_hbm, srel_hbm, nidx_hbm, cmb_hbm, out_hbm,
             srel_v, nbs, be0, be1, cb0, cb1, outb, sem0, sem1, si0, si1):
    wid = lax.axis_index("s") * NC + lax.axis_index("c")
    ibase = wid * NCHUNK
    obase = wid * ROWS_PER_W

    pltpu.sync_copy(srel_hbm, srel_v)
    pltpu.sync_copy(nidx_hbm.at[wid], nbs)

    def to_lines(k, carry):
        for g in range(GROUPS):
            sl = pl.ds(g * LANES, LANES)
            nbs[k, sl] = lax.shift_right_logical(nbs[k, sl], 2)
        return carry

    lax.fori_loop(0, NCHUNK, to_lines, 0)

    def stage_start(k, be, cb, sem, si):
        pltpu.async_copy(cmb_hbm.at[ibase + k], cb, si)
        pltpu.async_copy(ent_hbm.at[nbs.at[k]], be, sem)

    def process(k, be, cb, sem, si):
        pltpu.make_async_copy(cmb_hbm.at[ibase + k], cb, si).wait()
        pltpu.make_async_copy(ent_hbm.at[nbs.at[k]], be, sem).wait()
        for g in range(GROUPS):
            cvv = cb[pl.ds(g * LANES, LANES)]
            for i in range(LANES):
                r = g * LANES + i
                c = cvv[i]
                o = c & 127
                s = lax.shift_right_logical(c, 7)
                for chalf in range(0, D, LANES):
                    ev = be[r, pl.ds(o + chalf, LANES)]
                    rv = srel_v[pl.ds(s + chalf, LANES)]
                    outb[r, pl.ds(chalf, LANES)] = ev + rv
        pltpu.sync_copy(outb, out_hbm.at[pl.ds(obase + k * CHUNK, CHUNK)])

    stage_start(0, be0, cb0, sem0, si0)

    def pair(i, carry):
        k = i * 2
        stage_start(k + 1, be1, cb1, sem1, si1)
        process(k, be0, cb0, sem0, si0)

        @pl.when(k + 2 < NCHUNK)
        def _():
            stage_start(k + 2, be0, cb0, sem0, si0)

        process(k + 1, be1, cb1, sem1, si1)
        return carry

    lax.fori_loop(0, NCHUNK // 2, pair, 0)


@functools.partial(
    pl.kernel,
    out_type=jax.ShapeDtypeStruct((N, D), jnp.float32),
    mesh=plsc.VectorSubcoreMesh(
        core_axis_name="c", subcore_axis_name="s",
        num_cores=NC, num_subcores=NS),
    scratch_types=[
        pltpu.VMEM((2 * NUM_RELATIONS * D,), jnp.float32),
        pltpu.VMEM((NCHUNK, CHUNK), jnp.int32),
        pltpu.VMEM((CHUNK, 128), jnp.float32),
        pltpu.VMEM((CHUNK, 128), jnp.float32),
        pltpu.VMEM((CHUNK,), jnp.int32),
        pltpu.VMEM((CHUNK,), jnp.int32),
        pltpu.VMEM((CHUNK, D), jnp.float32),
        pltpu.SemaphoreType.DMA,
        pltpu.SemaphoreType.DMA,
        pltpu.SemaphoreType.DMA,
        pltpu.SemaphoreType.DMA,
    ],
)
def _transe_sc(ent_hbm, srel_hbm, nidx_hbm, cmb_hbm, out_hbm,
               srel_v, nbs, be0, be1, cb0, cb1, outb, sem0, sem1, si0, si1):
    _sc_body(ent_hbm, srel_hbm, nidx_hbm, cmb_hbm, out_hbm,
             srel_v, nbs, be0, be1, cb0, cb1, outb, sem0, sem1, si0, si1)


def kernel(entity_table, relation_table, neigh_nodes, neigh_rels, inverse):
    ent4 = entity_table.reshape(1000000 // PACK, 128)
    srel = jnp.concatenate([-relation_table, relation_table], axis=0).reshape(-1)
    n32 = neigh_nodes.astype(jnp.int32).reshape(NW, NCHUNK, CHUNK)
    lane_off = (neigh_nodes.astype(jnp.int32) & (PACK - 1)) << 5
    rel_off = (neigh_rels.astype(jnp.int32)
               + NUM_RELATIONS * inverse.astype(jnp.int32)) << 5
    cmb = ((rel_off << 7) | lane_off).reshape(NW * NCHUNK, CHUNK)
    out = _transe_sc(ent4, srel, n32, cmb)
    return out.reshape(B, L, D)
